# Initial kernel scaffold; baseline (speedup 1.0000x reference)
import jax
import jax.numpy as jnp
from jax.experimental import pallas as pl


def kernel(x, edge_index, batch, W_root1, W_rel1, b1, pw1, W_root2, W_rel2, b2, pw2):
    raise NotImplementedError("write your pallas kernel here")


# speedup vs baseline: 2.1375x; 2.1375x over previous
import jax
import jax.numpy as jnp
from jax.experimental import pallas as pl

N = 50000
E = 800000
G = 512
F_IN = 14
H = 128
RATIO = 0.8


def _graph_conv(x, src, dst, W_root, W_rel, b):
    msgs = jnp.take(x, src, axis=0)
    aggr = jnp.zeros_like(x).at[dst].add(msgs)
    return aggr @ W_rel.T + x @ W_root.T + b


def _topk_pool(x, pw, batch, node_mask, ratio, num_graphs):
    score = jnp.tanh((x @ pw) / jnp.linalg.norm(pw))
    score_key = jnp.where(node_mask, score, -2.0)
    order = jnp.lexsort((-score_key, batch))
    counts_all = jnp.bincount(batch, length=num_graphs)
    starts = jnp.cumsum(counts_all) - counts_all
    cnt = jax.ops.segment_sum(node_mask.astype(jnp.int32), batch, num_segments=num_graphs)
    k = jnp.ceil(ratio * cnt.astype(jnp.float32)).astype(jnp.int32)
    sorted_batch = batch[order]
    rank = jnp.arange(x.shape[0]) - starts[sorted_batch]
    sel = (rank < k[sorted_batch]) & node_mask[order]
    new_mask = jnp.zeros(x.shape[0], dtype=bool).at[order].set(sel)
    new_x = x * score[:, None] * new_mask[:, None].astype(x.dtype)
    return new_x, new_mask


def _masked_mean_max(x, batch, mask, num_graphs):
    m = mask.astype(x.dtype)[:, None]
    sums = jax.ops.segment_sum(x * m, batch, num_segments=num_graphs)
    cnt = jax.ops.segment_sum(mask.astype(x.dtype), batch, num_segments=num_graphs)
    mean = sums / jnp.maximum(cnt, 1.0)[:, None]
    xm = jnp.where(mask[:, None], x, -jnp.inf)
    mx = jax.ops.segment_max(xm, batch, num_segments=num_graphs)
    mx = jnp.where(jnp.isfinite(mx), mx, 0.0)
    return jnp.concatenate([mean, mx], axis=1)


def _add_kernel(a_ref, b_ref, o_ref):
    o_ref[...] = a_ref[...] + b_ref[...]


def kernel(x, edge_index, batch, W_root1, W_rel1, b1, pw1, W_root2, W_rel2, b2, pw2):
    src = edge_index[0]
    dst = edge_index[1]
    h = jax.nn.relu(_graph_conv(x, src, dst, W_root1, W_rel1, b1))
    h, mask1 = _topk_pool(h, pw1, batch, jnp.ones((N,), dtype=bool), RATIO, G)
    x1 = _masked_mean_max(h, batch, mask1, G)
    h2 = jax.nn.relu(_graph_conv(h, src, dst, W_root2, W_rel2, b2)) * mask1[:, None].astype(h.dtype)
    h2, mask2 = _topk_pool(h2, pw2, batch, mask1, RATIO, G)
    x2 = _masked_mean_max(h2, batch, mask2, G)
    return pl.pallas_call(
        _add_kernel,
        out_shape=jax.ShapeDtypeStruct((G, 2 * H), jnp.float32),
    )(x1, x2)


# speedup vs baseline: 4.0843x; 1.9107x over previous
import jax
import jax.numpy as jnp
from jax import lax
from jax.experimental import pallas as pl
from jax.experimental.pallas import tpu as pltpu
from jax.experimental.pallas import tpu_sc as plsc

N = 50000
E = 800000
G = 512
F_IN = 14
H = 128
RATIO = 0.8

NC = 2
NS = 16
CH = 128
E_PAD = 802816
NROW = 50048
EPT1 = E_PAD // (NC * NS)
NCH1 = EPT1 // CH
EPT2 = E_PAD // NS
NCH2 = EPT2 // CH
ZR = NROW // NS

_mesh = plsc.VectorSubcoreMesh(core_axis_name="c", subcore_axis_name="s")


def _aggr16_body(xp, src, dst, zrow, out, acc, sidx, didx, rows, sem):
    c = lax.axis_index("c")
    s = lax.axis_index("s")
    pltpu.sync_copy(zrow.at[pl.ds(s * ZR, ZR)], acc.at[pl.ds(s * ZR, ZR)])
    plsc.subcore_barrier()
    base = (c * NS + s) * EPT1

    def chunk(j, carry):
        off = base + j * CH
        pltpu.sync_copy(src.at[pl.ds(off, CH)], sidx)
        pltpu.sync_copy(dst.at[pl.ds(off, CH)], didx)
        pltpu.async_copy(xp.at[sidx], rows, sem).wait()
        pltpu.sync_copy(rows, acc.at[didx], add=True)
        return carry

    lax.fori_loop(0, NCH1, chunk, 0)
    plsc.subcore_barrier()
    pltpu.sync_copy(acc.at[pl.ds(s * ZR, ZR)], out.at[c, pl.ds(s * ZR, ZR)])


_aggr16 = pl.kernel(
    _aggr16_body,
    out_type=jax.ShapeDtypeStruct((NC, NROW, 16), jnp.float32),
    mesh=_mesh,
    compiler_params=pltpu.CompilerParams(use_tc_tiling_on_sc=False),
    scratch_types=[
        pltpu.VMEM_SHARED((NROW, 16), jnp.float32),
        pltpu.VMEM((CH,), jnp.int32),
        pltpu.VMEM((CH,), jnp.int32),
        pltpu.VMEM((CH, 16), jnp.float32),
        pltpu.SemaphoreType.DMA,
    ],
)


def _aggr32_body(h0, h1, h2, h3, src, dst, zrow, out, acc, sidx, didx, rows, sem):
    c = lax.axis_index("c")
    s = lax.axis_index("s")

    def one_pass(tab, slot):
        pltpu.sync_copy(zrow.at[pl.ds(s * ZR, ZR)], acc.at[pl.ds(s * ZR, ZR)])
        plsc.subcore_barrier()
        base = s * EPT2

        def chunk(j, carry):
            off = base + j * CH
            pltpu.sync_copy(src.at[pl.ds(off, CH)], sidx)
            pltpu.sync_copy(dst.at[pl.ds(off, CH)], didx)
            pltpu.async_copy(tab.at[sidx], rows, sem).wait()
            pltpu.sync_copy(rows, acc.at[didx], add=True)
            return carry

        lax.fori_loop(0, NCH2, chunk, 0)
        plsc.subcore_barrier()
        pltpu.sync_copy(acc.at[pl.ds(s * ZR, ZR)], out.at[slot, pl.ds(s * ZR, ZR)])
        plsc.subcore_barrier()

    @pl.when(c == 0)
    def _():
        one_pass(h0, 0)
        one_pass(h1, 1)

    @pl.when(c == 1)
    def _():
        one_pass(h2, 2)
        one_pass(h3, 3)


_aggr32 = pl.kernel(
    _aggr32_body,
    out_type=jax.ShapeDtypeStruct((4, NROW, 32), jnp.float32),
    mesh=_mesh,
    compiler_params=pltpu.CompilerParams(use_tc_tiling_on_sc=False),
    scratch_types=[
        pltpu.VMEM_SHARED((NROW, 32), jnp.float32),
        pltpu.VMEM((CH,), jnp.int32),
        pltpu.VMEM((CH,), jnp.int32),
        pltpu.VMEM((CH, 32), jnp.float32),
        pltpu.SemaphoreType.DMA,
    ],
)

BN = 400


def _mm(a, b):
    return jax.lax.dot_general(a, b, (((1,), (0,)), ((), ())),
                               precision=jax.lax.Precision.HIGHEST)


def _dense1_body(x_ref, p0_ref, p1_ref, a_ref, b_ref, bias_ref, pw_ref, h_ref, r_ref):
    h = _mm(x_ref[...], a_ref[...]) + _mm(p0_ref[...] + p1_ref[...], b_ref[...]) + bias_ref[...]
    h = jnp.maximum(h, 0.0)
    h_ref[...] = h
    r_ref[...] = jnp.tanh(_mm(h, pw_ref[...]))


def _dense2_body(x_ref, p_ref, a_ref, b_ref, bias_ref, pw_ref, m_ref, h_ref, r_ref):
    h = _mm(x_ref[...], a_ref[...]) + _mm(p_ref[...], b_ref[...]) + bias_ref[...]
    h = jnp.maximum(h, 0.0) * m_ref[...]
    h_ref[...] = h
    r_ref[...] = jnp.tanh(_mm(h, pw_ref[...]))


def _scale_body(h_ref, r_ref, m_ref, o_ref):
    o_ref[...] = h_ref[...] * r_ref[...] * m_ref[...]


def _add_body(a_ref, b_ref, o_ref):
    o_ref[...] = a_ref[...] + b_ref[...]


def _row_spec(w):
    return pl.BlockSpec((BN, w), lambda i: (i, 0))


def _full_spec(shape):
    return pl.BlockSpec(shape, lambda i: (0, 0))


def _dense1(xp, p0, p1, a, b, bias, pw):
    return pl.pallas_call(
        _dense1_body,
        grid=(N // BN,),
        in_specs=[
            _row_spec(16), _row_spec(16), _row_spec(16),
            _full_spec((16, H)), _full_spec((16, H)),
            _full_spec((1, H)), _full_spec((H, 1)),
        ],
        out_specs=[_row_spec(H), _row_spec(1)],
        out_shape=[
            jax.ShapeDtypeStruct((N, H), jnp.float32),
            jax.ShapeDtypeStruct((N, 1), jnp.float32),
        ],
    )(xp, p0, p1, a, b, bias, pw)


def _dense2(hp, p, a, b, bias, pw, m):
    return pl.pallas_call(
        _dense2_body,
        grid=(N // BN,),
        in_specs=[
            _row_spec(H), _row_spec(H),
            _full_spec((H, H)), _full_spec((H, H)),
            _full_spec((1, H)), _full_spec((H, 1)),
            _row_spec(1),
        ],
        out_specs=[_row_spec(H), _row_spec(1)],
        out_shape=[
            jax.ShapeDtypeStruct((N, H), jnp.float32),
            jax.ShapeDtypeStruct((N, 1), jnp.float32),
        ],
    )(hp, p, a, b, bias, pw, m)


def _scale(h, r, m):
    return pl.pallas_call(
        _scale_body,
        grid=(N // BN,),
        in_specs=[_row_spec(H), _row_spec(1), _row_spec(1)],
        out_specs=_row_spec(H),
        out_shape=jax.ShapeDtypeStruct((N, H), jnp.float32),
    )(h, r, m)


def _select(score, batch, node_mask):
    score_key = jnp.where(node_mask, score, -2.0)
    order = jnp.lexsort((-score_key, batch))
    counts_all = jnp.bincount(batch, length=G)
    starts = jnp.cumsum(counts_all) - counts_all
    cnt = jax.ops.segment_sum(node_mask.astype(jnp.int32), batch, num_segments=G)
    k = jnp.ceil(RATIO * cnt.astype(jnp.float32)).astype(jnp.int32)
    sorted_batch = batch[order]
    rank = jnp.arange(N) - starts[sorted_batch]
    sel = (rank < k[sorted_batch]) & node_mask[order]
    return jnp.zeros((N,), dtype=bool).at[order].set(sel)


def _masked_mean_max(x, batch, mask):
    m = mask.astype(x.dtype)[:, None]
    sums = jax.ops.segment_sum(x * m, batch, num_segments=G)
    cnt = jax.ops.segment_sum(mask.astype(x.dtype), batch, num_segments=G)
    mean = sums / jnp.maximum(cnt, 1.0)[:, None]
    xm = jnp.where(mask[:, None], x, -jnp.inf)
    mx = jax.ops.segment_max(xm, batch, num_segments=G)
    mx = jnp.where(jnp.isfinite(mx), mx, 0.0)
    return jnp.concatenate([mean, mx], axis=1)


def kernel(x, edge_index, batch, W_root1, W_rel1, b1, pw1, W_root2, W_rel2, b2, pw2):
    src = edge_index[0]
    dst = edge_index[1]
    pad = E_PAD - E
    srcp = jnp.concatenate([src, jnp.zeros((pad,), jnp.int32)])
    dstp = jnp.concatenate([dst, jnp.full((pad,), N, jnp.int32)])
    xp = jnp.pad(x, ((0, 0), (0, 16 - F_IN)))

    z16 = jnp.zeros((NROW, 16), jnp.float32)
    parts = _aggr16(xp, srcp, dstp, z16)[:, :N]

    a1 = jnp.pad(W_root1.T, ((0, 16 - F_IN), (0, 0)))
    b1m = jnp.pad(W_rel1.T, ((0, 16 - F_IN), (0, 0)))
    pw1n = (pw1 / jnp.linalg.norm(pw1))[:, None]
    h, r1 = _dense1(xp, parts[0], parts[1], a1, b1m, b1[None], pw1n)

    mask1 = _select(r1[:, 0], batch, jnp.ones((N,), dtype=bool))
    m1f = mask1.astype(jnp.float32)[:, None]
    hp = _scale(h, r1, m1f)

    h4 = hp.reshape(N, 4, 32).transpose(1, 0, 2)
    z32 = jnp.zeros((NROW, 32), jnp.float32)
    aggr4 = _aggr32(h4[0], h4[1], h4[2], h4[3], srcp, dstp, z32)[:, :N]
    aggr2 = aggr4.transpose(1, 0, 2).reshape(N, H)

    pw2n = (pw2 / jnp.linalg.norm(pw2))[:, None]
    h2, r2 = _dense2(hp, aggr2, W_root2.T, W_rel2.T, b2[None], pw2n, m1f)

    mask2 = _select(r2[:, 0], batch, mask1)
    h2p = _scale(h2, r2, mask2.astype(jnp.float32)[:, None])

    x1 = _masked_mean_max(hp, batch, mask1)
    x2 = _masked_mean_max(h2p, batch, mask2)
    return pl.pallas_call(
        _add_body,
        out_shape=jax.ShapeDtypeStruct((G, 2 * H), jnp.float32),
    )(x1, x2)


# speedup vs baseline: 4.4540x; 1.0905x over previous
import jax
import jax.numpy as jnp
from jax import lax
from jax.experimental import pallas as pl
from jax.experimental.pallas import tpu as pltpu
from jax.experimental.pallas import tpu_sc as plsc

N = 50000
E = 800000
G = 512
F_IN = 14
H = 128
RATIO = 0.8

NC = 2
NS = 16
CH = 128
E_PAD = 802816
NROW = 50048
EPT1 = E_PAD // (NC * NS)
NCH1 = EPT1 // CH
EPT2 = E_PAD // NS
NCH2 = EPT2 // CH
ZR = NROW // NS

_mesh = plsc.VectorSubcoreMesh(core_axis_name="c", subcore_axis_name="s")


def _sweep(src, dst, tab, acc, base, npair,
           sidx0, didx0, rows0, sem0, sidx1, didx1, rows1, sem1):
    def pair(m, carry):
        off0 = base + (2 * m) * CH
        off1 = off0 + CH
        pltpu.sync_copy(src.at[pl.ds(off0, CH)], sidx0)
        pltpu.sync_copy(dst.at[pl.ds(off0, CH)], didx0)
        g0 = pltpu.async_copy(tab.at[sidx0], rows0, sem0)
        pltpu.sync_copy(src.at[pl.ds(off1, CH)], sidx1)
        pltpu.sync_copy(dst.at[pl.ds(off1, CH)], didx1)
        g1 = pltpu.async_copy(tab.at[sidx1], rows1, sem1)
        g0.wait()
        pltpu.sync_copy(rows0, acc.at[didx0], add=True)
        g1.wait()
        pltpu.sync_copy(rows1, acc.at[didx1], add=True)
        return carry

    lax.fori_loop(0, npair, pair, 0)


def _aggr16_body(xp, src, dst, zrow, out, acc,
                 sidx0, didx0, rows0, sem0, sidx1, didx1, rows1, sem1):
    c = lax.axis_index("c")
    s = lax.axis_index("s")
    pltpu.sync_copy(zrow.at[pl.ds(s * ZR, ZR)], acc.at[pl.ds(s * ZR, ZR)])
    plsc.subcore_barrier()
    base = (c * NS + s) * EPT1
    _sweep(src, dst, xp, acc, base, NCH1 // 2,
           sidx0, didx0, rows0, sem0, sidx1, didx1, rows1, sem1)
    plsc.subcore_barrier()
    pltpu.sync_copy(acc.at[pl.ds(s * ZR, ZR)], out.at[c, pl.ds(s * ZR, ZR)])


_aggr16 = pl.kernel(
    _aggr16_body,
    out_type=jax.ShapeDtypeStruct((NC, NROW, 16), jnp.float32),
    mesh=_mesh,
    compiler_params=pltpu.CompilerParams(use_tc_tiling_on_sc=False),
    scratch_types=[
        pltpu.VMEM_SHARED((NROW, 16), jnp.float32),
        pltpu.VMEM((CH,), jnp.int32),
        pltpu.VMEM((CH,), jnp.int32),
        pltpu.VMEM((CH, 16), jnp.float32),
        pltpu.SemaphoreType.DMA,
        pltpu.VMEM((CH,), jnp.int32),
        pltpu.VMEM((CH,), jnp.int32),
        pltpu.VMEM((CH, 16), jnp.float32),
        pltpu.SemaphoreType.DMA,
    ],
)


def _aggr32_body(h0, h1, h2, h3, src, dst, zrow, out, acc,
                 sidx0, didx0, rows0, sem0, sidx1, didx1, rows1, sem1):
    c = lax.axis_index("c")
    s = lax.axis_index("s")

    def one_pass(tab, slot):
        pltpu.sync_copy(zrow.at[pl.ds(s * ZR, ZR)], acc.at[pl.ds(s * ZR, ZR)])
        plsc.subcore_barrier()
        base = s * EPT2
        _sweep(src, dst, tab, acc, base, NCH2 // 2,
               sidx0, didx0, rows0, sem0, sidx1, didx1, rows1, sem1)
        plsc.subcore_barrier()
        pltpu.sync_copy(acc.at[pl.ds(s * ZR, ZR)], out.at[slot, pl.ds(s * ZR, ZR)])
        plsc.subcore_barrier()

    @pl.when(c == 0)
    def _():
        one_pass(h0, 0)
        one_pass(h1, 1)

    @pl.when(c == 1)
    def _():
        one_pass(h2, 2)
        one_pass(h3, 3)


_aggr32 = pl.kernel(
    _aggr32_body,
    out_type=jax.ShapeDtypeStruct((4, NROW, 32), jnp.float32),
    mesh=_mesh,
    compiler_params=pltpu.CompilerParams(use_tc_tiling_on_sc=False),
    scratch_types=[
        pltpu.VMEM_SHARED((NROW, 32), jnp.float32),
        pltpu.VMEM((CH,), jnp.int32),
        pltpu.VMEM((CH,), jnp.int32),
        pltpu.VMEM((CH, 32), jnp.float32),
        pltpu.SemaphoreType.DMA,
        pltpu.VMEM((CH,), jnp.int32),
        pltpu.VMEM((CH,), jnp.int32),
        pltpu.VMEM((CH, 32), jnp.float32),
        pltpu.SemaphoreType.DMA,
    ],
)

BN = 400


def _mm(a, b):
    return jax.lax.dot_general(a, b, (((1,), (0,)), ((), ())),
                               precision=jax.lax.Precision.HIGHEST)


def _dense1_body(x_ref, p0_ref, p1_ref, a_ref, b_ref, bias_ref, pw_ref, h_ref, r_ref):
    h = _mm(x_ref[...], a_ref[...]) + _mm(p0_ref[...] + p1_ref[...], b_ref[...]) + bias_ref[...]
    h = jnp.maximum(h, 0.0)
    h_ref[...] = h
    r_ref[...] = jnp.tanh(_mm(h, pw_ref[...]))


def _dense2_body(x_ref, p_ref, a_ref, b_ref, bias_ref, pw_ref, m_ref, h_ref, r_ref):
    h = _mm(x_ref[...], a_ref[...]) + _mm(p_ref[...], b_ref[...]) + bias_ref[...]
    h = jnp.maximum(h, 0.0) * m_ref[...]
    h_ref[...] = h
    r_ref[...] = jnp.tanh(_mm(h, pw_ref[...]))


def _scale_body(h_ref, r_ref, m_ref, o_ref):
    o_ref[...] = h_ref[...] * r_ref[...] * m_ref[...]


def _add_body(a_ref, b_ref, o_ref):
    o_ref[...] = a_ref[...] + b_ref[...]


def _row_spec(w):
    return pl.BlockSpec((BN, w), lambda i: (i, 0))


def _full_spec(shape):
    return pl.BlockSpec(shape, lambda i: (0, 0))


def _dense1(xp, p0, p1, a, b, bias, pw):
    return pl.pallas_call(
        _dense1_body,
        grid=(N // BN,),
        in_specs=[
            _row_spec(16), _row_spec(16), _row_spec(16),
            _full_spec((16, H)), _full_spec((16, H)),
            _full_spec((1, H)), _full_spec((H, 1)),
        ],
        out_specs=[_row_spec(H), _row_spec(1)],
        out_shape=[
            jax.ShapeDtypeStruct((N, H), jnp.float32),
            jax.ShapeDtypeStruct((N, 1), jnp.float32),
        ],
    )(xp, p0, p1, a, b, bias, pw)


def _dense2(hp, p, a, b, bias, pw, m):
    return pl.pallas_call(
        _dense2_body,
        grid=(N // BN,),
        in_specs=[
            _row_spec(H), _row_spec(H),
            _full_spec((H, H)), _full_spec((H, H)),
            _full_spec((1, H)), _full_spec((H, 1)),
            _row_spec(1),
        ],
        out_specs=[_row_spec(H), _row_spec(1)],
        out_shape=[
            jax.ShapeDtypeStruct((N, H), jnp.float32),
            jax.ShapeDtypeStruct((N, 1), jnp.float32),
        ],
    )(hp, p, a, b, bias, pw, m)


def _scale(h, r, m):
    return pl.pallas_call(
        _scale_body,
        grid=(N // BN,),
        in_specs=[_row_spec(H), _row_spec(1), _row_spec(1)],
        out_specs=_row_spec(H),
        out_shape=jax.ShapeDtypeStruct((N, H), jnp.float32),
    )(h, r, m)


def _select(score, batch, node_mask):
    score_key = jnp.where(node_mask, score, -2.0)
    order = jnp.lexsort((-score_key, batch))
    counts_all = jnp.bincount(batch, length=G)
    starts = jnp.cumsum(counts_all) - counts_all
    cnt = jax.ops.segment_sum(node_mask.astype(jnp.int32), batch, num_segments=G)
    k = jnp.ceil(RATIO * cnt.astype(jnp.float32)).astype(jnp.int32)
    sorted_batch = batch[order]
    rank = jnp.arange(N) - starts[sorted_batch]
    sel = (rank < k[sorted_batch]) & node_mask[order]
    return jnp.zeros((N,), dtype=bool).at[order].set(sel)


def _masked_mean_max(x, batch, mask):
    m = mask.astype(x.dtype)[:, None]
    sums = jax.ops.segment_sum(x * m, batch, num_segments=G)
    cnt = jax.ops.segment_sum(mask.astype(x.dtype), batch, num_segments=G)
    mean = sums / jnp.maximum(cnt, 1.0)[:, None]
    xm = jnp.where(mask[:, None], x, -jnp.inf)
    mx = jax.ops.segment_max(xm, batch, num_segments=G)
    mx = jnp.where(jnp.isfinite(mx), mx, 0.0)
    return jnp.concatenate([mean, mx], axis=1)


def kernel(x, edge_index, batch, W_root1, W_rel1, b1, pw1, W_root2, W_rel2, b2, pw2):
    src = edge_index[0]
    dst = edge_index[1]
    pad = E_PAD - E
    srcp = jnp.concatenate([src, jnp.zeros((pad,), jnp.int32)])
    dstp = jnp.concatenate([dst, jnp.full((pad,), N, jnp.int32)])
    xp = jnp.pad(x, ((0, 0), (0, 16 - F_IN)))

    z16 = jnp.zeros((NROW, 16), jnp.float32)
    parts = _aggr16(xp, srcp, dstp, z16)[:, :N]

    a1 = jnp.pad(W_root1.T, ((0, 16 - F_IN), (0, 0)))
    b1m = jnp.pad(W_rel1.T, ((0, 16 - F_IN), (0, 0)))
    pw1n = (pw1 / jnp.linalg.norm(pw1))[:, None]
    h, r1 = _dense1(xp, parts[0], parts[1], a1, b1m, b1[None], pw1n)

    mask1 = _select(r1[:, 0], batch, jnp.ones((N,), dtype=bool))
    m1f = mask1.astype(jnp.float32)[:, None]
    hp = _scale(h, r1, m1f)

    h4 = hp.reshape(N, 4, 32).transpose(1, 0, 2)
    z32 = jnp.zeros((NROW, 32), jnp.float32)
    aggr4 = _aggr32(h4[0], h4[1], h4[2], h4[3], srcp, dstp, z32)[:, :N]
    aggr2 = aggr4.transpose(1, 0, 2).reshape(N, H)

    pw2n = (pw2 / jnp.linalg.norm(pw2))[:, None]
    h2, r2 = _dense2(hp, aggr2, W_root2.T, W_rel2.T, b2[None], pw2n, m1f)

    mask2 = _select(r2[:, 0], batch, mask1)
    h2p = _scale(h2, r2, mask2.astype(jnp.float32)[:, None])

    x1 = _masked_mean_max(hp, batch, mask1)
    x2 = _masked_mean_max(h2p, batch, mask2)
    return pl.pallas_call(
        _add_body,
        out_shape=jax.ShapeDtypeStruct((G, 2 * H), jnp.float32),
    )(x1, x2)
